# baseline (speedup 1.0000x reference)
import functools

import jax
import jax.numpy as jnp
from jax import lax
from jax.experimental import pallas as pl
from jax.experimental.pallas import tpu as pltpu
from jax.experimental.pallas import tpu_sc as plsc

_FFT = 16384
_SPEC_TYPE = 2
_FREQ_SCALE = 2000000.0 / (2.0 * _FFT)
_NPEAKS = 10
_NN = 5
_NTRAIN = 1000000
_NCLS = 21
_DIM = 20

_NC = 2
_NS = 16
_NW = _NC * _NS
_RPW = _NTRAIN // _NW
_CHUNK = 1250
_NCHUNKS = _RPW // _CHUNK
_GROUPS = (_CHUNK + 15) // 16


def _feat_body(spec_ref, bg_ref, out_ref):
    spec = spec_ref[...]
    bg = jnp.mean(bg_ref[...], axis=0, keepdims=True)
    cl = spec - bg
    pos_iota = lax.broadcasted_iota(jnp.int32, (1, _FFT), 1)
    lane128 = lax.broadcasted_iota(jnp.int32, (1, 128), 1)
    big = jnp.int32(1 << 30)
    feat = jnp.zeros((1, 128), jnp.float32)
    for i in range(_NPEAKS):
        m = jnp.max(cl)
        pos = jnp.min(jnp.where(cl == m, pos_iota, big))
        feat = jnp.where(lane128 == i, m, feat)
        feat = jnp.where(lane128 == (i + _NPEAKS),
                         pos.astype(jnp.float32) * _FREQ_SCALE, feat)
        cl = jnp.where(pos_iota == pos, -jnp.inf, cl)
    out_ref[...] = feat


def _knn_body(xt_ref, feat_ref, outd_ref, outi_ref, buf, featv, odv, oiv):
    wid = lax.axis_index("s") * _NC + lax.axis_index("c")
    base = wid * _RPW
    pltpu.sync_copy(feat_ref, featv)
    f_lo = featv[0:16]
    f_hi = featv[16:32]
    fs = [f_lo[d] for d in range(16)] + [f_hi[d] for d in range(_DIM - 16)]
    iota16 = lax.iota(jnp.int32, 16)
    inf16 = jnp.full((16,), jnp.inf, jnp.float32)

    def chunk_body(c, carry):
        off = pl.multiple_of(base + c * _CHUNK, 8)
        pltpu.sync_copy(xt_ref.at[pl.ds(off, _CHUNK), :], buf)
        gbase = base + c * _CHUNK

        def group_body(g, gcarry):
            bd, bi, thv = gcarry
            rows = g * 16 + iota16
            valid = rows < _CHUNK
            rc = jnp.minimum(rows, _CHUNK - 1)
            acc = jnp.zeros((16,), jnp.float32)
            for d in range(_DIM):
                v = plsc.load_gather(buf, [rc, jnp.full((16,), d, jnp.int32)])
                t = v - fs[d]
                acc = acc + t * t
            acc = jnp.where(valid, acc, inf16)
            hit = jnp.any(acc < thv)

            def slow(args):
                sbd, sbi, _ = args
                gidx = gbase + rows
                nd, ni = plsc.sort_key_val(acc, gidx)
                ndr = lax.rev(nd, (0,))
                nir = lax.rev(ni, (0,))
                take_a = sbd <= ndr
                md = jnp.where(take_a, sbd, ndr)
                mi = jnp.where(take_a, sbi, nir)
                bd2, bi2 = plsc.sort_key_val(md, mi)
                thv2 = jnp.full((16,), bd2[_NN - 1])
                return bd2, bi2, thv2

            return lax.cond(hit, slow, lambda a: a, (bd, bi, thv))

        return lax.fori_loop(0, _GROUPS, group_body, carry)

    init = (inf16, jnp.zeros((16,), jnp.int32), inf16)
    bd, bi, _ = lax.fori_loop(0, _NCHUNKS, chunk_body, init)
    odv[...] = bd
    oiv[...] = bi
    pltpu.sync_copy(odv, outd_ref.at[wid])
    pltpu.sync_copy(oiv, outi_ref.at[wid])


_knn_call = functools.partial(
    pl.kernel,
    mesh=plsc.VectorSubcoreMesh(core_axis_name="c", subcore_axis_name="s"),
    out_type=[jax.ShapeDtypeStruct((_NW, 16), jnp.float32),
              jax.ShapeDtypeStruct((_NW, 16), jnp.int32)],
    scratch_types=[pltpu.VMEM((_CHUNK, _DIM), jnp.float32),
                   pltpu.VMEM((32,), jnp.float32),
                   pltpu.VMEM((16,), jnp.float32),
                   pltpu.VMEM((16,), jnp.int32)],
    compiler_params=pltpu.CompilerParams(needs_layout_passes=False,
                                         use_tc_tiling_on_sc=False),
)(_knn_body)


def _merge_body(cd_ref, ci_ref, idx_ref, d0_ref):
    cd = cd_ref[...]
    cif = ci_ref[...].astype(jnp.float32)
    fp = (lax.broadcasted_iota(jnp.int32, (_NW, 16), 0) * 16
          + lax.broadcasted_iota(jnp.int32, (_NW, 16), 1))
    big = jnp.int32(1 << 30)
    lane16 = lax.broadcasted_iota(jnp.int32, (1, 16), 1)
    idxv = jnp.zeros((1, 16), jnp.int32)
    d0sq = jnp.float32(0.0)
    for k in range(_NN):
        m = jnp.min(cd)
        if k == 0:
            d0sq = m
        pos = jnp.min(jnp.where(cd == m, fp, big))
        pmask = fp == pos
        idx = jnp.sum(jnp.where(pmask, cif, 0.0)).astype(jnp.int32)
        idxv = jnp.where(lane16 == k, idx, idxv)
        cd = jnp.where(pmask, jnp.inf, cd)
    idx_ref[...] = idxv
    d0_ref[...] = jnp.full((1, 16), d0sq, jnp.float32)


def _vote_body(rows_ref, apl_ref, sv_ref, ap_ref, d0_ref, out_ref):
    votes = jnp.sum(rows_ref[...], axis=0, keepdims=True)
    lane21 = lax.broadcasted_iota(jnp.int32, (1, _NCLS), 1)
    big = jnp.int32(1 << 30)
    vm = jnp.max(votes)
    cls = jnp.min(jnp.where(votes == vm, lane21, big))
    d0sq = d0_ref[...][0, 0]
    cls = jnp.where(d0sq > 100.0, jnp.int32(2 * _NPEAKS), cls)

    lane16 = lax.broadcasted_iota(jnp.int32, (1, 16), 1)
    sv = sv_ref[...]
    apl = apl_ref[...]
    ap = ap_ref[0, 0]
    is_on = cls < _NPEAKS
    is_off = (cls >= _NPEAKS) & (cls < 2 * _NPEAKS)
    idx_on = jnp.clip(cls, 0, _NPEAKS - 1)
    idx_off = jnp.clip(cls - _NPEAKS, 0, _NPEAKS - 1)
    ap_on = jnp.sum(jnp.where(lane16 == idx_on, apl, 0.0))
    sv_on = jnp.where(lane16 == idx_on, ap_on, sv)
    sv_off = jnp.where(lane16 == idx_off, 0.0, sv)
    nsv = jnp.where(is_on, sv_on, jnp.where(is_off, sv_off, sv))
    known = jnp.sum(jnp.where(lane16 < _NPEAKS, nsv, 0.0))
    nsv = jnp.where(lane16 == _NPEAKS, ap - known, nsv)
    out_ref[...] = nsv


def kernel(X, X_train, y_train, background_vector, apparent_power_list,
           current_state_vector):
    spec = X[_SPEC_TYPE * _FFT:(_SPEC_TYPE + 1) * _FFT].reshape(1, _FFT)
    ap = X[-2:-1]

    feat128 = pl.pallas_call(
        _feat_body,
        out_shape=jax.ShapeDtypeStruct((1, 128), jnp.float32),
    )(spec, background_vector)
    feat32 = feat128[0, :32]

    cand_d, cand_i = _knn_call(X_train, feat32)

    idx16, d016 = pl.pallas_call(
        _merge_body,
        out_shape=[jax.ShapeDtypeStruct((1, 16), jnp.int32),
                   jax.ShapeDtypeStruct((1, 16), jnp.float32)],
    )(cand_d, cand_i)

    rows5 = jnp.take(y_train, idx16[0, :_NN], axis=0)

    apl16 = jnp.pad(apparent_power_list, (0, 6)).reshape(1, 16)
    sv16 = jnp.pad(current_state_vector, (0, 5)).reshape(1, 16)

    out16 = pl.pallas_call(
        _vote_body,
        out_shape=jax.ShapeDtypeStruct((1, 16), jnp.float32),
        in_specs=[pl.BlockSpec(memory_space=pltpu.VMEM),
                  pl.BlockSpec(memory_space=pltpu.VMEM),
                  pl.BlockSpec(memory_space=pltpu.VMEM),
                  pl.BlockSpec(memory_space=pltpu.SMEM),
                  pl.BlockSpec(memory_space=pltpu.VMEM)],
        out_specs=pl.BlockSpec(memory_space=pltpu.VMEM),
    )(rows5, apl16, sv16, ap.reshape(1, 1), d016)

    return out16[0, :11]


# speedup vs baseline: 4.4621x; 4.4621x over previous
import functools

import jax
import jax.numpy as jnp
from jax import lax
from jax.experimental import pallas as pl
from jax.experimental.pallas import tpu as pltpu
from jax.experimental.pallas import tpu_sc as plsc

_FFT = 16384
_SPEC_TYPE = 2
_FREQ_SCALE = 2000000.0 / (2.0 * _FFT)
_NPEAKS = 10
_NN = 5
_NTRAIN = 1000000
_NCLS = 21
_DIM = 20

_B = 4096
_NB = 245
_NTOT = _NB * _B
_NC = 2
_NS = 16
_NW = _NC * _NS
_RPW = _NTOT // _NW
_GROUPS = _RPW // 16


def _feat_body(spec_ref, bg_ref, out_ref):
    spec = spec_ref[...]
    bg = jnp.mean(bg_ref[...], axis=0, keepdims=True)
    cl = spec - bg
    pos_iota = lax.broadcasted_iota(jnp.int32, (1, _FFT), 1)
    lane128 = lax.broadcasted_iota(jnp.int32, (1, 128), 1)
    big = jnp.int32(1 << 30)
    feat = jnp.zeros((1, 128), jnp.float32)
    for i in range(_NPEAKS):
        m = jnp.max(cl)
        pos = jnp.min(jnp.where(cl == m, pos_iota, big))
        feat = jnp.where(lane128 == i, m, feat)
        feat = jnp.where(lane128 == (i + _NPEAKS),
                         pos.astype(jnp.float32) * _FREQ_SCALE, feat)
        cl = jnp.where(pos_iota == pos, -jnp.inf, cl)
    out_ref[...] = feat


def _dist_body(featc_ref, xtt_ref, out_ref):
    x = xtt_ref[...]
    f = featc_ref[...]
    diff = x - f
    sq = diff * diff
    b = sq[0:8] + sq[8:16]
    t2 = jnp.concatenate(
        [sq[16:20], jnp.zeros((4, _B), jnp.float32)], axis=0)
    c = b + t2
    d1 = c[0:4] + c[4:8]
    d2 = d1[0:2] + d1[2:4]
    d3 = d2[0:1] + d2[1:2]
    dist = jnp.sqrt(d3)
    i = pl.program_id(0)
    glob = i * _B + lax.broadcasted_iota(jnp.int32, (1, _B), 1)
    dist = jnp.where(glob < _NTRAIN, dist, jnp.inf)
    out_ref[...] = dist.reshape(_B)


def _topk_body(d_ref, outd_ref, outi_ref, buf, odv, oiv):
    wid = lax.axis_index("s") * _NC + lax.axis_index("c")
    base = wid * _RPW
    pltpu.sync_copy(d_ref.at[pl.ds(pl.multiple_of(base, 8), _RPW)], buf)
    iota16 = lax.iota(jnp.int32, 16)
    inf16 = jnp.full((16,), jnp.inf, jnp.float32)

    def group_body(g, carry):
        bd, bi, thv = carry
        v = buf[pl.ds(g * 16, 16)]
        hit = jnp.any(v < thv)

        def slow(args):
            sbd, sbi, _ = args
            gidx = base + g * 16 + iota16
            nd, ni = plsc.sort_key_val(v, gidx)
            ndr = lax.rev(nd, (0,))
            nir = lax.rev(ni, (0,))
            take_a = (sbd < ndr) | ((sbd == ndr) & (sbi < nir))
            md = jnp.where(take_a, sbd, ndr)
            mi = jnp.where(take_a, sbi, nir)
            bd2, bi2 = plsc.sort_key_val(md, mi)
            thv2 = jnp.full((16,), bd2[_NN - 1])
            return bd2, bi2, thv2

        return lax.cond(hit, slow, lambda a: a, (bd, bi, thv))

    init = (inf16, jnp.zeros((16,), jnp.int32), inf16)
    bd, bi, _ = lax.fori_loop(0, _GROUPS, group_body, init)
    odv[...] = bd
    oiv[...] = bi
    pltpu.sync_copy(odv, outd_ref.at[wid])
    pltpu.sync_copy(oiv, outi_ref.at[wid])


_topk_call = functools.partial(
    pl.kernel,
    mesh=plsc.VectorSubcoreMesh(core_axis_name="c", subcore_axis_name="s"),
    out_type=[jax.ShapeDtypeStruct((_NW, 16), jnp.float32),
              jax.ShapeDtypeStruct((_NW, 16), jnp.int32)],
    scratch_types=[pltpu.VMEM((_RPW,), jnp.float32),
                   pltpu.VMEM((16,), jnp.float32),
                   pltpu.VMEM((16,), jnp.int32)],
    compiler_params=pltpu.CompilerParams(needs_layout_passes=False),
)(_topk_body)


def _merge_body(cd_ref, ci_ref, idx_ref, d0_ref):
    cd = cd_ref[...]
    ci = ci_ref[...]
    cif = ci.astype(jnp.float32)
    big = jnp.int32(1 << 30)
    biginf = jnp.float32(3.0e38)
    lane16 = lax.broadcasted_iota(jnp.int32, (1, 16), 1)
    idxv = jnp.zeros((1, 16), jnp.int32)
    d0 = jnp.float32(0.0)
    for k in range(_NN):
        m = jnp.min(cd)
        if k == 0:
            d0 = m
        idx = jnp.min(jnp.where(cd == m, ci, big))
        idxf = idx.astype(jnp.float32)
        pmask = (cd == m) & (cif == idxf)
        idxv = jnp.where(lane16 == k, idx, idxv)
        cd = jnp.where(pmask, biginf, cd)
    idx_ref[...] = idxv
    d0_ref[...] = jnp.full((1, 16), d0, jnp.float32)


def _vote_body(rows_ref, apl_ref, sv_ref, ap_ref, d0_ref, out_ref):
    votes = jnp.sum(rows_ref[...], axis=0, keepdims=True)
    lane21 = lax.broadcasted_iota(jnp.int32, (1, _NCLS), 1)
    big = jnp.int32(1 << 30)
    vm = jnp.max(votes)
    cls = jnp.min(jnp.where(votes == vm, lane21, big))
    d0 = d0_ref[...][0, 0]
    cls = jnp.where(d0 > 10.0, jnp.int32(2 * _NPEAKS), cls)

    lane16 = lax.broadcasted_iota(jnp.int32, (1, 16), 1)
    sv = sv_ref[...]
    apl = apl_ref[...]
    ap = ap_ref[0, 0]
    is_on = cls < _NPEAKS
    is_off = (cls >= _NPEAKS) & (cls < 2 * _NPEAKS)
    idx_on = jnp.clip(cls, 0, _NPEAKS - 1)
    idx_off = jnp.clip(cls - _NPEAKS, 0, _NPEAKS - 1)
    ap_on = jnp.sum(jnp.where(lane16 == idx_on, apl, 0.0))
    sv_on = jnp.where(lane16 == idx_on, ap_on, sv)
    sv_off = jnp.where(lane16 == idx_off, 0.0, sv)
    nsv = jnp.where(is_on, sv_on, jnp.where(is_off, sv_off, sv))
    known = jnp.sum(jnp.where(lane16 < _NPEAKS, nsv, 0.0))
    nsv = jnp.where(lane16 == _NPEAKS, ap - known, nsv)
    out_ref[...] = nsv


def kernel(X, X_train, y_train, background_vector, apparent_power_list,
           current_state_vector):
    spec = X[_SPEC_TYPE * _FFT:(_SPEC_TYPE + 1) * _FFT].reshape(1, _FFT)
    ap = X[-2:-1]

    feat128 = pl.pallas_call(
        _feat_body,
        out_shape=jax.ShapeDtypeStruct((1, 128), jnp.float32),
    )(spec, background_vector)
    featc = feat128[0, :_DIM].reshape(_DIM, 1)

    xtt = X_train.T

    dists = pl.pallas_call(
        _dist_body,
        grid=(_NB,),
        in_specs=[pl.BlockSpec((_DIM, 1), lambda i: (0, 0)),
                  pl.BlockSpec((_DIM, _B), lambda i: (0, i))],
        out_specs=pl.BlockSpec((_B,), lambda i: (i,)),
        out_shape=jax.ShapeDtypeStruct((_NTOT,), jnp.float32),
    )(featc, xtt)

    cand_d, cand_i = _topk_call(dists)

    idx16, d016 = pl.pallas_call(
        _merge_body,
        out_shape=[jax.ShapeDtypeStruct((1, 16), jnp.int32),
                   jax.ShapeDtypeStruct((1, 16), jnp.float32)],
    )(cand_d, cand_i)

    rows5 = jnp.take(y_train, idx16[0, :_NN], axis=0)

    apl16 = jnp.pad(apparent_power_list, (0, 6)).reshape(1, 16)
    sv16 = jnp.pad(current_state_vector, (0, 5)).reshape(1, 16)

    out16 = pl.pallas_call(
        _vote_body,
        out_shape=jax.ShapeDtypeStruct((1, 16), jnp.float32),
        in_specs=[pl.BlockSpec(memory_space=pltpu.VMEM),
                  pl.BlockSpec(memory_space=pltpu.VMEM),
                  pl.BlockSpec(memory_space=pltpu.VMEM),
                  pl.BlockSpec(memory_space=pltpu.SMEM),
                  pl.BlockSpec(memory_space=pltpu.VMEM)],
        out_specs=pl.BlockSpec(memory_space=pltpu.VMEM),
    )(rows5, apl16, sv16, ap.reshape(1, 1), d016)

    return out16[0, :11]


# speedup vs baseline: 7.8947x; 1.7693x over previous
import functools

import jax
import jax.numpy as jnp
from jax import lax
from jax.experimental import pallas as pl
from jax.experimental.pallas import tpu as pltpu
from jax.experimental.pallas import tpu_sc as plsc

_FFT = 16384
_SPEC_TYPE = 2
_FREQ_SCALE = 2000000.0 / (2.0 * _FFT)
_NPEAKS = 10
_NN = 5
_NTRAIN = 1000000
_NCLS = 21
_DIM = 20

_B = 8192
_NB = 123
_NTOT = _NB * _B
_NC = 2
_NS = 16
_NW = _NC * _NS
_RPW = _NTOT // _NW
_GROUPS = _RPW // 16
_G8 = _GROUPS // 8


def _feat_body(spec_ref, bg_ref, out_ref):
    spec = spec_ref[...]
    bg = jnp.mean(bg_ref[...], axis=0, keepdims=True)
    cl = spec - bg
    pos_iota = lax.broadcasted_iota(jnp.int32, (1, _FFT), 1)
    lane128 = lax.broadcasted_iota(jnp.int32, (1, 128), 1)
    big = jnp.int32(1 << 30)
    feat = jnp.zeros((1, 128), jnp.float32)
    for i in range(_NPEAKS):
        m = jnp.max(cl)
        pos = jnp.min(jnp.where(cl == m, pos_iota, big))
        feat = jnp.where(lane128 == i, m, feat)
        feat = jnp.where(lane128 == (i + _NPEAKS),
                         pos.astype(jnp.float32) * _FREQ_SCALE, feat)
        cl = jnp.where(pos_iota == pos, -jnp.inf, cl)
    out_ref[...] = feat


def _dist_body(featc_ref, xtt_ref, out_ref):
    x = xtt_ref[...]
    f = featc_ref[...]
    diff = x - f
    sq = diff * diff
    b = sq[0:8] + sq[8:16]
    t2 = jnp.concatenate(
        [sq[16:20], jnp.zeros((4, _B), jnp.float32)], axis=0)
    c = b + t2
    d1 = c[0:4] + c[4:8]
    d2 = d1[0:2] + d1[2:4]
    d3 = d2[0:1] + d2[1:2]
    dist = jnp.sqrt(d3)
    i = pl.program_id(0)
    glob = i * _B + lax.broadcasted_iota(jnp.int32, (1, _B), 1)
    dist = jnp.where(glob < _NTRAIN, dist, jnp.inf)
    out_ref[...] = dist.reshape(_B)


def _topk_body(d_ref, outd_ref, outi_ref, buf, odv, oiv):
    wid = lax.axis_index("s") * _NC + lax.axis_index("c")
    base = wid * _RPW
    pltpu.sync_copy(d_ref.at[pl.ds(pl.multiple_of(base, 8), _RPW)], buf)
    iota16 = lax.iota(jnp.int32, 16)
    inf16 = jnp.full((16,), jnp.inf, jnp.float32)

    def merge_group(carry, v, gidx):
        hit = jnp.any(v < carry[2])

        def slow(args):
            sbd, sbi, _ = args
            nd, ni = plsc.sort_key_val(v, gidx)
            ndr = lax.rev(nd, (0,))
            nir = lax.rev(ni, (0,))
            take_a = (sbd < ndr) | ((sbd == ndr) & (sbi < nir))
            md = jnp.where(take_a, sbd, ndr)
            mi = jnp.where(take_a, sbi, nir)
            bd2, bi2 = plsc.sort_key_val(md, mi)
            thv2 = jnp.full((16,), bd2[_NN - 1])
            return bd2, bi2, thv2

        return lax.cond(hit, slow, lambda a: a, carry)

    def group8_body(g8, carry):
        thv = carry[2]
        vs = [buf[pl.ds((g8 * 8 + j) * 16, 16)] for j in range(8)]
        h = vs[0] < thv
        for j in range(1, 8):
            h = h | (vs[j] < thv)
        hit = jnp.any(h)

        def slow(args):
            c = args
            for j in range(8):
                c = merge_group(c, vs[j], base + (g8 * 8 + j) * 16 + iota16)
            return c

        return lax.cond(hit, slow, lambda a: a, carry)

    init = (inf16, jnp.zeros((16,), jnp.int32), inf16)
    bd, bi, _ = lax.fori_loop(0, _G8, group8_body, init)
    odv[...] = bd
    oiv[...] = bi
    pltpu.sync_copy(odv, outd_ref.at[wid])
    pltpu.sync_copy(oiv, outi_ref.at[wid])


_topk_call = functools.partial(
    pl.kernel,
    mesh=plsc.VectorSubcoreMesh(core_axis_name="c", subcore_axis_name="s"),
    out_type=[jax.ShapeDtypeStruct((_NW, 16), jnp.float32),
              jax.ShapeDtypeStruct((_NW, 16), jnp.int32)],
    scratch_types=[pltpu.VMEM((_RPW,), jnp.float32),
                   pltpu.VMEM((16,), jnp.float32),
                   pltpu.VMEM((16,), jnp.int32)],
    compiler_params=pltpu.CompilerParams(needs_layout_passes=False),
)(_topk_body)


def _merge_body(cd_ref, ci_ref, idx_ref, d0_ref):
    cd = cd_ref[...]
    ci = ci_ref[...]
    cif = ci.astype(jnp.float32)
    big = jnp.int32(1 << 30)
    biginf = jnp.float32(3.0e38)
    lane16 = lax.broadcasted_iota(jnp.int32, (1, 16), 1)
    idxv = jnp.zeros((1, 16), jnp.int32)
    d0 = jnp.float32(0.0)
    for k in range(_NN):
        m = jnp.min(cd)
        if k == 0:
            d0 = m
        idx = jnp.min(jnp.where(cd == m, ci, big))
        idxf = idx.astype(jnp.float32)
        pmask = (cd == m) & (cif == idxf)
        idxv = jnp.where(lane16 == k, idx, idxv)
        cd = jnp.where(pmask, biginf, cd)
    idx_ref[...] = idxv
    d0_ref[...] = jnp.full((1, 16), d0, jnp.float32)


def _vote_body(rows_ref, apl_ref, sv_ref, ap_ref, d0_ref, out_ref):
    votes = jnp.sum(rows_ref[...], axis=0, keepdims=True)
    lane21 = lax.broadcasted_iota(jnp.int32, (1, _NCLS), 1)
    big = jnp.int32(1 << 30)
    vm = jnp.max(votes)
    cls = jnp.min(jnp.where(votes == vm, lane21, big))
    d0 = d0_ref[...][0, 0]
    cls = jnp.where(d0 > 10.0, jnp.int32(2 * _NPEAKS), cls)

    lane16 = lax.broadcasted_iota(jnp.int32, (1, 16), 1)
    sv = sv_ref[...]
    apl = apl_ref[...]
    ap = ap_ref[0, 0]
    is_on = cls < _NPEAKS
    is_off = (cls >= _NPEAKS) & (cls < 2 * _NPEAKS)
    idx_on = jnp.clip(cls, 0, _NPEAKS - 1)
    idx_off = jnp.clip(cls - _NPEAKS, 0, _NPEAKS - 1)
    ap_on = jnp.sum(jnp.where(lane16 == idx_on, apl, 0.0))
    sv_on = jnp.where(lane16 == idx_on, ap_on, sv)
    sv_off = jnp.where(lane16 == idx_off, 0.0, sv)
    nsv = jnp.where(is_on, sv_on, jnp.where(is_off, sv_off, sv))
    known = jnp.sum(jnp.where(lane16 < _NPEAKS, nsv, 0.0))
    nsv = jnp.where(lane16 == _NPEAKS, ap - known, nsv)
    out_ref[...] = nsv


def kernel(X, X_train, y_train, background_vector, apparent_power_list,
           current_state_vector):
    spec = X[_SPEC_TYPE * _FFT:(_SPEC_TYPE + 1) * _FFT].reshape(1, _FFT)
    ap = X[-2:-1]

    feat128 = pl.pallas_call(
        _feat_body,
        out_shape=jax.ShapeDtypeStruct((1, 128), jnp.float32),
    )(spec, background_vector)
    featc = feat128[0, :_DIM].reshape(_DIM, 1)

    xtt = X_train.T

    dists = pl.pallas_call(
        _dist_body,
        grid=(_NB,),
        in_specs=[pl.BlockSpec((_DIM, 1), lambda i: (0, 0)),
                  pl.BlockSpec((_DIM, _B), lambda i: (0, i))],
        out_specs=pl.BlockSpec((_B,), lambda i: (i,)),
        out_shape=jax.ShapeDtypeStruct((_NTOT,), jnp.float32),
    )(featc, xtt)

    cand_d, cand_i = _topk_call(dists)

    idx16, d016 = pl.pallas_call(
        _merge_body,
        out_shape=[jax.ShapeDtypeStruct((1, 16), jnp.int32),
                   jax.ShapeDtypeStruct((1, 16), jnp.float32)],
    )(cand_d, cand_i)

    rows5 = jnp.take(y_train, idx16[0, :_NN], axis=0)

    apl16 = jnp.pad(apparent_power_list, (0, 6)).reshape(1, 16)
    sv16 = jnp.pad(current_state_vector, (0, 5)).reshape(1, 16)

    out16 = pl.pallas_call(
        _vote_body,
        out_shape=jax.ShapeDtypeStruct((1, 16), jnp.float32),
        in_specs=[pl.BlockSpec(memory_space=pltpu.VMEM),
                  pl.BlockSpec(memory_space=pltpu.VMEM),
                  pl.BlockSpec(memory_space=pltpu.VMEM),
                  pl.BlockSpec(memory_space=pltpu.SMEM),
                  pl.BlockSpec(memory_space=pltpu.VMEM)],
        out_specs=pl.BlockSpec(memory_space=pltpu.VMEM),
    )(rows5, apl16, sv16, ap.reshape(1, 1), d016)

    return out16[0, :11]


# speedup vs baseline: 9.6146x; 1.2179x over previous
import functools

import jax
import jax.numpy as jnp
from jax import lax
from jax.experimental import pallas as pl
from jax.experimental.pallas import tpu as pltpu
from jax.experimental.pallas import tpu_sc as plsc

_FFT = 16384
_SPEC_TYPE = 2
_FREQ_SCALE = 2000000.0 / (2.0 * _FFT)
_NPEAKS = 10
_NN = 5
_NTRAIN = 1000000
_NCLS = 21
_DIM = 20

_B = 16384
_NB = 62
_NTOT = _NB * _B
_NC = 2
_NS = 16
_NW = _NC * _NS
_RPW = _NTOT // _NW
_GROUPS = _RPW // 16
_G8 = _GROUPS // 8


def _feat_body(spec_ref, bg_ref, out_ref):
    spec = spec_ref[...]
    bg = jnp.mean(bg_ref[...], axis=0, keepdims=True)
    cl = spec - bg
    pos_iota = lax.broadcasted_iota(jnp.int32, (1, _FFT), 1)
    lane128 = lax.broadcasted_iota(jnp.int32, (1, 128), 1)
    big = jnp.int32(1 << 30)
    feat = jnp.zeros((1, 128), jnp.float32)
    for i in range(_NPEAKS):
        m = jnp.max(cl)
        pos = jnp.min(jnp.where(cl == m, pos_iota, big))
        feat = jnp.where(lane128 == i, m, feat)
        feat = jnp.where(lane128 == (i + _NPEAKS),
                         pos.astype(jnp.float32) * _FREQ_SCALE, feat)
        cl = jnp.where(pos_iota == pos, -jnp.inf, cl)
    out_ref[...] = feat


def _dist_body(featc_ref, xtt_ref, out_ref):
    x = xtt_ref[...]
    f = featc_ref[...]
    diff = x - f
    sq = diff * diff
    b = sq[0:8] + sq[8:16]
    t2 = jnp.concatenate(
        [sq[16:20], jnp.zeros((4, _B), jnp.float32)], axis=0)
    c = b + t2
    d1 = c[0:4] + c[4:8]
    d2 = d1[0:2] + d1[2:4]
    d3 = d2[0:1] + d2[1:2]
    dist = jnp.sqrt(d3)
    i = pl.program_id(0)
    glob = i * _B + lax.broadcasted_iota(jnp.int32, (1, _B), 1)
    dist = jnp.where(glob < _NTRAIN, dist, jnp.inf)
    out_ref[...] = dist.reshape(_B)


def _topk_body(d_ref, outd_ref, outi_ref, buf, odv, oiv):
    wid = lax.axis_index("s") * _NC + lax.axis_index("c")
    base = wid * _RPW
    pltpu.sync_copy(d_ref.at[pl.ds(pl.multiple_of(base, 8), _RPW)], buf)
    iota16 = lax.iota(jnp.int32, 16)
    inf16 = jnp.full((16,), jnp.inf, jnp.float32)

    def merge_group(carry, v, gidx):
        hit = jnp.any(v < carry[2])

        def slow(args):
            sbd, sbi, _ = args
            nd, ni = plsc.sort_key_val(v, gidx)
            ndr = lax.rev(nd, (0,))
            nir = lax.rev(ni, (0,))
            take_a = (sbd < ndr) | ((sbd == ndr) & (sbi < nir))
            md = jnp.where(take_a, sbd, ndr)
            mi = jnp.where(take_a, sbi, nir)
            bd2, bi2 = plsc.sort_key_val(md, mi)
            thv2 = jnp.full((16,), bd2[_NN - 1])
            return bd2, bi2, thv2

        return lax.cond(hit, slow, lambda a: a, carry)

    def group8_body(g8, carry):
        thv = carry[2]
        vs = [buf[pl.ds((g8 * 8 + j) * 16, 16)] for j in range(8)]
        h = vs[0] < thv
        for j in range(1, 8):
            h = h | (vs[j] < thv)
        hit = jnp.any(h)

        def slow(args):
            c = args
            for j in range(8):
                c = merge_group(c, vs[j], base + (g8 * 8 + j) * 16 + iota16)
            return c

        return lax.cond(hit, slow, lambda a: a, carry)

    init = (inf16, jnp.zeros((16,), jnp.int32), inf16)
    bd, bi, _ = lax.fori_loop(0, _G8, group8_body, init)
    odv[...] = bd
    oiv[...] = bi
    pltpu.sync_copy(odv, outd_ref.at[wid])
    pltpu.sync_copy(oiv, outi_ref.at[wid])


_topk_call = functools.partial(
    pl.kernel,
    mesh=plsc.VectorSubcoreMesh(core_axis_name="c", subcore_axis_name="s"),
    out_type=[jax.ShapeDtypeStruct((_NW, 16), jnp.float32),
              jax.ShapeDtypeStruct((_NW, 16), jnp.int32)],
    scratch_types=[pltpu.VMEM((_RPW,), jnp.float32),
                   pltpu.VMEM((16,), jnp.float32),
                   pltpu.VMEM((16,), jnp.int32)],
    compiler_params=pltpu.CompilerParams(needs_layout_passes=False),
)(_topk_body)


def _merge_body(cd_ref, ci_ref, idx_ref, d0_ref):
    cd = cd_ref[...]
    ci = ci_ref[...]
    cif = ci.astype(jnp.float32)
    big = jnp.int32(1 << 30)
    biginf = jnp.float32(3.0e38)
    lane16 = lax.broadcasted_iota(jnp.int32, (1, 16), 1)
    idxv = jnp.zeros((1, 16), jnp.int32)
    d0 = jnp.float32(0.0)
    for k in range(_NN):
        m = jnp.min(cd)
        if k == 0:
            d0 = m
        idx = jnp.min(jnp.where(cd == m, ci, big))
        idxf = idx.astype(jnp.float32)
        pmask = (cd == m) & (cif == idxf)
        idxv = jnp.where(lane16 == k, idx, idxv)
        cd = jnp.where(pmask, biginf, cd)
    idx_ref[...] = idxv
    d0_ref[...] = jnp.full((1, 16), d0, jnp.float32)


def _vote_body(rows_ref, apl_ref, sv_ref, ap_ref, d0_ref, out_ref):
    votes = jnp.sum(rows_ref[...], axis=0, keepdims=True)
    lane21 = lax.broadcasted_iota(jnp.int32, (1, _NCLS), 1)
    big = jnp.int32(1 << 30)
    vm = jnp.max(votes)
    cls = jnp.min(jnp.where(votes == vm, lane21, big))
    d0 = d0_ref[...][0, 0]
    cls = jnp.where(d0 > 10.0, jnp.int32(2 * _NPEAKS), cls)

    lane16 = lax.broadcasted_iota(jnp.int32, (1, 16), 1)
    sv = sv_ref[...]
    apl = apl_ref[...]
    ap = ap_ref[0, 0]
    is_on = cls < _NPEAKS
    is_off = (cls >= _NPEAKS) & (cls < 2 * _NPEAKS)
    idx_on = jnp.clip(cls, 0, _NPEAKS - 1)
    idx_off = jnp.clip(cls - _NPEAKS, 0, _NPEAKS - 1)
    ap_on = jnp.sum(jnp.where(lane16 == idx_on, apl, 0.0))
    sv_on = jnp.where(lane16 == idx_on, ap_on, sv)
    sv_off = jnp.where(lane16 == idx_off, 0.0, sv)
    nsv = jnp.where(is_on, sv_on, jnp.where(is_off, sv_off, sv))
    known = jnp.sum(jnp.where(lane16 < _NPEAKS, nsv, 0.0))
    nsv = jnp.where(lane16 == _NPEAKS, ap - known, nsv)
    out_ref[...] = nsv


def kernel(X, X_train, y_train, background_vector, apparent_power_list,
           current_state_vector):
    spec = X[_SPEC_TYPE * _FFT:(_SPEC_TYPE + 1) * _FFT].reshape(1, _FFT)
    ap = X[-2:-1]

    feat128 = pl.pallas_call(
        _feat_body,
        out_shape=jax.ShapeDtypeStruct((1, 128), jnp.float32),
    )(spec, background_vector)
    featc = feat128[0, :_DIM].reshape(_DIM, 1)

    xtt = X_train.T

    dists = pl.pallas_call(
        _dist_body,
        grid=(_NB,),
        in_specs=[pl.BlockSpec((_DIM, 1), lambda i: (0, 0)),
                  pl.BlockSpec((_DIM, _B), lambda i: (0, i))],
        out_specs=pl.BlockSpec((_B,), lambda i: (i,)),
        out_shape=jax.ShapeDtypeStruct((_NTOT,), jnp.float32),
    )(featc, xtt)

    cand_d, cand_i = _topk_call(dists)

    idx16, d016 = pl.pallas_call(
        _merge_body,
        out_shape=[jax.ShapeDtypeStruct((1, 16), jnp.int32),
                   jax.ShapeDtypeStruct((1, 16), jnp.float32)],
    )(cand_d, cand_i)

    rows5 = jnp.take(y_train, idx16[0, :_NN], axis=0)

    apl16 = jnp.pad(apparent_power_list, (0, 6)).reshape(1, 16)
    sv16 = jnp.pad(current_state_vector, (0, 5)).reshape(1, 16)

    out16 = pl.pallas_call(
        _vote_body,
        out_shape=jax.ShapeDtypeStruct((1, 16), jnp.float32),
        in_specs=[pl.BlockSpec(memory_space=pltpu.VMEM),
                  pl.BlockSpec(memory_space=pltpu.VMEM),
                  pl.BlockSpec(memory_space=pltpu.VMEM),
                  pl.BlockSpec(memory_space=pltpu.SMEM),
                  pl.BlockSpec(memory_space=pltpu.VMEM)],
        out_specs=pl.BlockSpec(memory_space=pltpu.VMEM),
    )(rows5, apl16, sv16, ap.reshape(1, 1), d016)

    return out16[0, :11]


# speedup vs baseline: 10.9024x; 1.1339x over previous
import functools

import jax
import jax.numpy as jnp
from jax import lax
from jax.experimental import pallas as pl
from jax.experimental.pallas import tpu as pltpu
from jax.experimental.pallas import tpu_sc as plsc

_FFT = 16384
_SPEC_TYPE = 2
_FREQ_SCALE = 2000000.0 / (2.0 * _FFT)
_NPEAKS = 10
_NN = 5
_NTRAIN = 1000000
_NCLS = 21
_DIM = 20

_B = 32768
_NB = 31
_NTOT = _NB * _B
_NC = 2
_NS = 16
_NW = _NC * _NS
_RPW = _NTOT // _NW
_GROUPS = _RPW // 16
_G8 = _GROUPS // 8


def _feat_body(spec_ref, bg_ref, out_ref):
    spec = spec_ref[...]
    bg = jnp.mean(bg_ref[...], axis=0, keepdims=True)
    cl = spec - bg
    pos_iota = lax.broadcasted_iota(jnp.int32, (1, _FFT), 1)
    lane128 = lax.broadcasted_iota(jnp.int32, (1, 128), 1)
    big = jnp.int32(1 << 30)
    feat = jnp.zeros((1, 128), jnp.float32)
    for i in range(_NPEAKS):
        m = jnp.max(cl)
        pos = jnp.min(jnp.where(cl == m, pos_iota, big))
        feat = jnp.where(lane128 == i, m, feat)
        feat = jnp.where(lane128 == (i + _NPEAKS),
                         pos.astype(jnp.float32) * _FREQ_SCALE, feat)
        cl = jnp.where(pos_iota == pos, -jnp.inf, cl)
    out_ref[...] = feat


def _dist_body(featc_ref, xtt_ref, out_ref):
    x = xtt_ref[...]
    f = featc_ref[...]
    diff = x - f
    sq = diff * diff
    b = sq[0:8] + sq[8:16]
    t2 = jnp.concatenate(
        [sq[16:20], jnp.zeros((4, _B), jnp.float32)], axis=0)
    c = b + t2
    d1 = c[0:4] + c[4:8]
    d2 = d1[0:2] + d1[2:4]
    d3 = d2[0:1] + d2[1:2]
    dist = jnp.sqrt(d3)
    i = pl.program_id(0)
    glob = i * _B + lax.broadcasted_iota(jnp.int32, (1, _B), 1)
    dist = jnp.where(glob < _NTRAIN, dist, jnp.inf)
    out_ref[...] = dist.reshape(_B)


def _topk_body(d_ref, outd_ref, outi_ref, buf, odv, oiv):
    wid = lax.axis_index("s") * _NC + lax.axis_index("c")
    base = wid * _RPW
    pltpu.sync_copy(d_ref.at[pl.ds(pl.multiple_of(base, 8), _RPW)], buf)
    iota16 = lax.iota(jnp.int32, 16)
    inf16 = jnp.full((16,), jnp.inf, jnp.float32)

    def merge_group(carry, v, gidx):
        hit = jnp.any(v < carry[2])

        def slow(args):
            sbd, sbi, _ = args
            nd, ni = plsc.sort_key_val(v, gidx)
            ndr = lax.rev(nd, (0,))
            nir = lax.rev(ni, (0,))
            take_a = (sbd < ndr) | ((sbd == ndr) & (sbi < nir))
            md = jnp.where(take_a, sbd, ndr)
            mi = jnp.where(take_a, sbi, nir)
            bd2, bi2 = plsc.sort_key_val(md, mi)
            thv2 = jnp.full((16,), bd2[_NN - 1])
            return bd2, bi2, thv2

        return lax.cond(hit, slow, lambda a: a, carry)

    def group8_body(g8, carry):
        thv = carry[2]
        vs = [buf[pl.ds((g8 * 8 + j) * 16, 16)] for j in range(8)]
        h = vs[0] < thv
        for j in range(1, 8):
            h = h | (vs[j] < thv)
        hit = jnp.any(h)

        def slow(args):
            c = args
            for j in range(8):
                c = merge_group(c, vs[j], base + (g8 * 8 + j) * 16 + iota16)
            return c

        return lax.cond(hit, slow, lambda a: a, carry)

    init = (inf16, jnp.zeros((16,), jnp.int32), inf16)
    bd, bi, _ = lax.fori_loop(0, _G8, group8_body, init)
    odv[...] = bd
    oiv[...] = bi
    pltpu.sync_copy(odv, outd_ref.at[wid])
    pltpu.sync_copy(oiv, outi_ref.at[wid])


_topk_call = functools.partial(
    pl.kernel,
    mesh=plsc.VectorSubcoreMesh(core_axis_name="c", subcore_axis_name="s"),
    out_type=[jax.ShapeDtypeStruct((_NW, 16), jnp.float32),
              jax.ShapeDtypeStruct((_NW, 16), jnp.int32)],
    scratch_types=[pltpu.VMEM((_RPW,), jnp.float32),
                   pltpu.VMEM((16,), jnp.float32),
                   pltpu.VMEM((16,), jnp.int32)],
    compiler_params=pltpu.CompilerParams(needs_layout_passes=False),
)(_topk_body)


def _merge_body(cd_ref, ci_ref, idx_ref, d0_ref):
    cd = cd_ref[...]
    ci = ci_ref[...]
    cif = ci.astype(jnp.float32)
    big = jnp.int32(1 << 30)
    biginf = jnp.float32(3.0e38)
    lane16 = lax.broadcasted_iota(jnp.int32, (1, 16), 1)
    idxv = jnp.zeros((1, 16), jnp.int32)
    d0 = jnp.float32(0.0)
    for k in range(_NN):
        m = jnp.min(cd)
        if k == 0:
            d0 = m
        idx = jnp.min(jnp.where(cd == m, ci, big))
        idxf = idx.astype(jnp.float32)
        pmask = (cd == m) & (cif == idxf)
        idxv = jnp.where(lane16 == k, idx, idxv)
        cd = jnp.where(pmask, biginf, cd)
    idx_ref[...] = idxv
    d0_ref[...] = jnp.full((1, 16), d0, jnp.float32)


def _vote_body(rows_ref, apl_ref, sv_ref, ap_ref, d0_ref, out_ref):
    votes = jnp.sum(rows_ref[...], axis=0, keepdims=True)
    lane21 = lax.broadcasted_iota(jnp.int32, (1, _NCLS), 1)
    big = jnp.int32(1 << 30)
    vm = jnp.max(votes)
    cls = jnp.min(jnp.where(votes == vm, lane21, big))
    d0 = d0_ref[...][0, 0]
    cls = jnp.where(d0 > 10.0, jnp.int32(2 * _NPEAKS), cls)

    lane16 = lax.broadcasted_iota(jnp.int32, (1, 16), 1)
    sv = sv_ref[...]
    apl = apl_ref[...]
    ap = ap_ref[0, 0]
    is_on = cls < _NPEAKS
    is_off = (cls >= _NPEAKS) & (cls < 2 * _NPEAKS)
    idx_on = jnp.clip(cls, 0, _NPEAKS - 1)
    idx_off = jnp.clip(cls - _NPEAKS, 0, _NPEAKS - 1)
    ap_on = jnp.sum(jnp.where(lane16 == idx_on, apl, 0.0))
    sv_on = jnp.where(lane16 == idx_on, ap_on, sv)
    sv_off = jnp.where(lane16 == idx_off, 0.0, sv)
    nsv = jnp.where(is_on, sv_on, jnp.where(is_off, sv_off, sv))
    known = jnp.sum(jnp.where(lane16 < _NPEAKS, nsv, 0.0))
    nsv = jnp.where(lane16 == _NPEAKS, ap - known, nsv)
    out_ref[...] = nsv


def kernel(X, X_train, y_train, background_vector, apparent_power_list,
           current_state_vector):
    spec = X[_SPEC_TYPE * _FFT:(_SPEC_TYPE + 1) * _FFT].reshape(1, _FFT)
    ap = X[-2:-1]

    feat128 = pl.pallas_call(
        _feat_body,
        out_shape=jax.ShapeDtypeStruct((1, 128), jnp.float32),
    )(spec, background_vector)
    featc = feat128[0, :_DIM].reshape(_DIM, 1)

    xtt = X_train.T

    dists = pl.pallas_call(
        _dist_body,
        grid=(_NB,),
        in_specs=[pl.BlockSpec((_DIM, 1), lambda i: (0, 0)),
                  pl.BlockSpec((_DIM, _B), lambda i: (0, i))],
        out_specs=pl.BlockSpec((_B,), lambda i: (i,)),
        out_shape=jax.ShapeDtypeStruct((_NTOT,), jnp.float32),
    )(featc, xtt)

    cand_d, cand_i = _topk_call(dists)

    idx16, d016 = pl.pallas_call(
        _merge_body,
        out_shape=[jax.ShapeDtypeStruct((1, 16), jnp.int32),
                   jax.ShapeDtypeStruct((1, 16), jnp.float32)],
    )(cand_d, cand_i)

    rows5 = jnp.take(y_train, idx16[0, :_NN], axis=0)

    apl16 = jnp.pad(apparent_power_list, (0, 6)).reshape(1, 16)
    sv16 = jnp.pad(current_state_vector, (0, 5)).reshape(1, 16)

    out16 = pl.pallas_call(
        _vote_body,
        out_shape=jax.ShapeDtypeStruct((1, 16), jnp.float32),
        in_specs=[pl.BlockSpec(memory_space=pltpu.VMEM),
                  pl.BlockSpec(memory_space=pltpu.VMEM),
                  pl.BlockSpec(memory_space=pltpu.VMEM),
                  pl.BlockSpec(memory_space=pltpu.SMEM),
                  pl.BlockSpec(memory_space=pltpu.VMEM)],
        out_specs=pl.BlockSpec(memory_space=pltpu.VMEM),
    )(rows5, apl16, sv16, ap.reshape(1, 1), d016)

    return out16[0, :11]


# speedup vs baseline: 11.2486x; 1.0318x over previous
import functools

import jax
import jax.numpy as jnp
from jax import lax
from jax.experimental import pallas as pl
from jax.experimental.pallas import tpu as pltpu
from jax.experimental.pallas import tpu_sc as plsc

_FFT = 16384
_SPEC_TYPE = 2
_FREQ_SCALE = 2000000.0 / (2.0 * _FFT)
_NPEAKS = 10
_NN = 5
_NTRAIN = 1000000
_NCLS = 21
_DIM = 20

_B = 65536
_NB = 16
_NTOT = _NB * _B
_NC = 2
_NS = 16
_NW = _NC * _NS
_RPW = _NTOT // _NW
_GROUPS = _RPW // 16
_G8 = _GROUPS // 8


def _feat_body(spec_ref, bg_ref, out_ref):
    spec = spec_ref[...]
    bg = jnp.mean(bg_ref[...], axis=0, keepdims=True)
    cl = spec - bg
    pos_iota = lax.broadcasted_iota(jnp.int32, (1, _FFT), 1)
    lane128 = lax.broadcasted_iota(jnp.int32, (1, 128), 1)
    big = jnp.int32(1 << 30)
    feat = jnp.zeros((1, 128), jnp.float32)
    for i in range(_NPEAKS):
        m = jnp.max(cl)
        pos = jnp.min(jnp.where(cl == m, pos_iota, big))
        feat = jnp.where(lane128 == i, m, feat)
        feat = jnp.where(lane128 == (i + _NPEAKS),
                         pos.astype(jnp.float32) * _FREQ_SCALE, feat)
        cl = jnp.where(pos_iota == pos, -jnp.inf, cl)
    out_ref[...] = feat


def _dist_body(featc_ref, xtt_ref, out_ref):
    x = xtt_ref[...]
    f = featc_ref[...]
    diff = x - f
    sq = diff * diff
    b = sq[0:8] + sq[8:16]
    t2 = jnp.concatenate(
        [sq[16:20], jnp.zeros((4, _B), jnp.float32)], axis=0)
    c = b + t2
    d1 = c[0:4] + c[4:8]
    d2 = d1[0:2] + d1[2:4]
    d3 = d2[0:1] + d2[1:2]
    dist = jnp.sqrt(d3)
    i = pl.program_id(0)
    glob = i * _B + lax.broadcasted_iota(jnp.int32, (1, _B), 1)
    dist = jnp.where(glob < _NTRAIN, dist, jnp.inf)
    out_ref[...] = dist.reshape(_B)


def _topk_body(d_ref, outd_ref, outi_ref, buf, odv, oiv):
    wid = lax.axis_index("s") * _NC + lax.axis_index("c")
    base = wid * _RPW
    pltpu.sync_copy(d_ref.at[pl.ds(pl.multiple_of(base, 8), _RPW)], buf)
    iota16 = lax.iota(jnp.int32, 16)
    inf16 = jnp.full((16,), jnp.inf, jnp.float32)

    def merge_group(carry, v, gidx):
        hit = jnp.any(v < carry[2])

        def slow(args):
            sbd, sbi, _ = args
            nd, ni = plsc.sort_key_val(v, gidx)
            ndr = lax.rev(nd, (0,))
            nir = lax.rev(ni, (0,))
            take_a = (sbd < ndr) | ((sbd == ndr) & (sbi < nir))
            md = jnp.where(take_a, sbd, ndr)
            mi = jnp.where(take_a, sbi, nir)
            bd2, bi2 = plsc.sort_key_val(md, mi)
            thv2 = jnp.full((16,), bd2[_NN - 1])
            return bd2, bi2, thv2

        return lax.cond(hit, slow, lambda a: a, carry)

    def group8_body(g8, carry):
        thv = carry[2]
        vs = [buf[pl.ds((g8 * 8 + j) * 16, 16)] for j in range(8)]
        h = vs[0] < thv
        for j in range(1, 8):
            h = h | (vs[j] < thv)
        hit = jnp.any(h)

        def slow(args):
            c = args
            for j in range(8):
                c = merge_group(c, vs[j], base + (g8 * 8 + j) * 16 + iota16)
            return c

        return lax.cond(hit, slow, lambda a: a, carry)

    init = (inf16, jnp.zeros((16,), jnp.int32), inf16)
    bd, bi, _ = lax.fori_loop(0, _G8, group8_body, init)
    odv[...] = bd
    oiv[...] = bi
    pltpu.sync_copy(odv, outd_ref.at[wid])
    pltpu.sync_copy(oiv, outi_ref.at[wid])


_topk_call = functools.partial(
    pl.kernel,
    mesh=plsc.VectorSubcoreMesh(core_axis_name="c", subcore_axis_name="s"),
    out_type=[jax.ShapeDtypeStruct((_NW, 16), jnp.float32),
              jax.ShapeDtypeStruct((_NW, 16), jnp.int32)],
    scratch_types=[pltpu.VMEM((_RPW,), jnp.float32),
                   pltpu.VMEM((16,), jnp.float32),
                   pltpu.VMEM((16,), jnp.int32)],
    compiler_params=pltpu.CompilerParams(needs_layout_passes=False),
)(_topk_body)


def _merge_body(cd_ref, ci_ref, idx_ref, d0_ref):
    cd = cd_ref[...]
    ci = ci_ref[...]
    cif = ci.astype(jnp.float32)
    big = jnp.int32(1 << 30)
    biginf = jnp.float32(3.0e38)
    lane16 = lax.broadcasted_iota(jnp.int32, (1, 16), 1)
    idxv = jnp.zeros((1, 16), jnp.int32)
    d0 = jnp.float32(0.0)
    for k in range(_NN):
        m = jnp.min(cd)
        if k == 0:
            d0 = m
        idx = jnp.min(jnp.where(cd == m, ci, big))
        idxf = idx.astype(jnp.float32)
        pmask = (cd == m) & (cif == idxf)
        idxv = jnp.where(lane16 == k, idx, idxv)
        cd = jnp.where(pmask, biginf, cd)
    idx_ref[...] = idxv
    d0_ref[...] = jnp.full((1, 16), d0, jnp.float32)


def _vote_body(rows_ref, apl_ref, sv_ref, ap_ref, d0_ref, out_ref):
    votes = jnp.sum(rows_ref[...], axis=0, keepdims=True)
    lane21 = lax.broadcasted_iota(jnp.int32, (1, _NCLS), 1)
    big = jnp.int32(1 << 30)
    vm = jnp.max(votes)
    cls = jnp.min(jnp.where(votes == vm, lane21, big))
    d0 = d0_ref[...][0, 0]
    cls = jnp.where(d0 > 10.0, jnp.int32(2 * _NPEAKS), cls)

    lane16 = lax.broadcasted_iota(jnp.int32, (1, 16), 1)
    sv = sv_ref[...]
    apl = apl_ref[...]
    ap = ap_ref[0, 0]
    is_on = cls < _NPEAKS
    is_off = (cls >= _NPEAKS) & (cls < 2 * _NPEAKS)
    idx_on = jnp.clip(cls, 0, _NPEAKS - 1)
    idx_off = jnp.clip(cls - _NPEAKS, 0, _NPEAKS - 1)
    ap_on = jnp.sum(jnp.where(lane16 == idx_on, apl, 0.0))
    sv_on = jnp.where(lane16 == idx_on, ap_on, sv)
    sv_off = jnp.where(lane16 == idx_off, 0.0, sv)
    nsv = jnp.where(is_on, sv_on, jnp.where(is_off, sv_off, sv))
    known = jnp.sum(jnp.where(lane16 < _NPEAKS, nsv, 0.0))
    nsv = jnp.where(lane16 == _NPEAKS, ap - known, nsv)
    out_ref[...] = nsv


def kernel(X, X_train, y_train, background_vector, apparent_power_list,
           current_state_vector):
    spec = X[_SPEC_TYPE * _FFT:(_SPEC_TYPE + 1) * _FFT].reshape(1, _FFT)
    ap = X[-2:-1]

    feat128 = pl.pallas_call(
        _feat_body,
        out_shape=jax.ShapeDtypeStruct((1, 128), jnp.float32),
    )(spec, background_vector)
    featc = feat128[0, :_DIM].reshape(_DIM, 1)

    xtt = X_train.T

    dists = pl.pallas_call(
        _dist_body,
        grid=(_NB,),
        in_specs=[pl.BlockSpec((_DIM, 1), lambda i: (0, 0)),
                  pl.BlockSpec((_DIM, _B), lambda i: (0, i))],
        out_specs=pl.BlockSpec((_B,), lambda i: (i,)),
        out_shape=jax.ShapeDtypeStruct((_NTOT,), jnp.float32),
    )(featc, xtt)

    cand_d, cand_i = _topk_call(dists)

    idx16, d016 = pl.pallas_call(
        _merge_body,
        out_shape=[jax.ShapeDtypeStruct((1, 16), jnp.int32),
                   jax.ShapeDtypeStruct((1, 16), jnp.float32)],
    )(cand_d, cand_i)

    rows5 = jnp.take(y_train, idx16[0, :_NN], axis=0)

    apl16 = jnp.pad(apparent_power_list, (0, 6)).reshape(1, 16)
    sv16 = jnp.pad(current_state_vector, (0, 5)).reshape(1, 16)

    out16 = pl.pallas_call(
        _vote_body,
        out_shape=jax.ShapeDtypeStruct((1, 16), jnp.float32),
        in_specs=[pl.BlockSpec(memory_space=pltpu.VMEM),
                  pl.BlockSpec(memory_space=pltpu.VMEM),
                  pl.BlockSpec(memory_space=pltpu.VMEM),
                  pl.BlockSpec(memory_space=pltpu.SMEM),
                  pl.BlockSpec(memory_space=pltpu.VMEM)],
        out_specs=pl.BlockSpec(memory_space=pltpu.VMEM),
    )(rows5, apl16, sv16, ap.reshape(1, 1), d016)

    return out16[0, :11]
